# baseline (speedup 1.0000x reference)
import functools

import jax
import jax.numpy as jnp
from jax import lax
from jax.experimental import pallas as pl
from jax.experimental.pallas import tpu as pltpu
from jax.experimental.pallas import tpu_sc as plsc

B = 128
D = 128
NW = 16
CHUNK = B // NW


@functools.cache
def _build_sc_gather():
    mesh = plsc.VectorSubcoreMesh(
        core_axis_name="c", subcore_axis_name="s", num_cores=1)

    @functools.partial(
        pl.kernel,
        mesh=mesh,
        out_type=jax.ShapeDtypeStruct((3 * B, D), jnp.float32),
        scratch_types=[
            pltpu.VMEM((CHUNK,), jnp.int32),
            pltpu.VMEM((CHUNK,), jnp.int32),
            pltpu.VMEM((CHUNK,), jnp.int32),
            pltpu.VMEM((CHUNK, D), jnp.float32),
            pltpu.VMEM((CHUNK, D), jnp.float32),
            pltpu.VMEM((CHUNK, D), jnp.float32),
            pltpu.SemaphoreType.DMA,
        ],
    )
    def _sc_gather(words, pos, neg, emb, oemb, out,
                   iw_v, ip_v, in_v, rw_v, rp_v, rn_v, sem):
        wid = lax.axis_index("s")
        base = wid * CHUNK
        pltpu.sync_copy(words.at[pl.ds(base, CHUNK)], iw_v)
        pltpu.sync_copy(pos.at[pl.ds(base, CHUNK)], ip_v)
        pltpu.sync_copy(neg.at[pl.ds(base, CHUNK)], in_v)
        cw = pltpu.make_async_copy(emb.at[iw_v], rw_v, sem)
        cp = pltpu.make_async_copy(oemb.at[ip_v], rp_v, sem)
        cn = pltpu.make_async_copy(oemb.at[in_v], rn_v, sem)
        cw.start()
        cp.start()
        cn.start()
        cw.wait()
        cp.wait()
        cn.wait()
        pltpu.sync_copy(rw_v, out.at[pl.ds(base, CHUNK)])
        pltpu.sync_copy(rp_v, out.at[pl.ds(B + base, CHUNK)])
        pltpu.sync_copy(rn_v, out.at[pl.ds(2 * B + base, CHUNK)])

    return _sc_gather


def _tc_loss_body(g_ref, out_ref):
    t = g_ref[0:B, :]
    p = g_ref[B:2 * B, :]
    n = g_ref[2 * B:3 * B, :]
    tt = t.T
    s_pos = jnp.sum(tt * p)
    s_neg = jnp.sum(tt * n)
    r = lax.broadcasted_iota(jnp.int32, (8, 128), 0)
    c = lax.broadcasted_iota(jnp.int32, (8, 128), 1)
    ma = ((r == 0) & (c == 0)).astype(jnp.float32)
    mb = ((r == 0) & (c == 1)).astype(jnp.float32)
    v = s_pos * ma - s_neg * mb
    ls = jnp.minimum(v, 0.0) - jnp.log1p(jnp.exp(-jnp.abs(v)))
    out_ref[...] = -jnp.sum(ls * (ma + mb))


def kernel(words, pos_contexts, neg_contexts, emb, out_emb):
    g = _build_sc_gather()(words, pos_contexts, neg_contexts, emb, out_emb)
    loss = pl.pallas_call(
        _tc_loss_body,
        out_shape=jax.ShapeDtypeStruct((), jnp.float32),
        out_specs=pl.BlockSpec(memory_space=pltpu.SMEM),
    )(g)
    return loss


# speedup vs baseline: 1.0490x; 1.0490x over previous
import functools

import jax
import jax.numpy as jnp
from jax import lax
from jax.experimental import pallas as pl
from jax.experimental.pallas import tpu as pltpu
from jax.experimental.pallas import tpu_sc as plsc

B = 128
D = 128
NW = 16
CHUNK = B // NW


@functools.cache
def _build_sc_gather():
    mesh = plsc.VectorSubcoreMesh(
        core_axis_name="c", subcore_axis_name="s", num_cores=1)

    @functools.partial(
        pl.kernel,
        mesh=mesh,
        out_type=jax.ShapeDtypeStruct((3 * B, D), jnp.float32),
        scratch_types=[
            pltpu.VMEM((CHUNK,), jnp.int32),
            pltpu.VMEM((CHUNK,), jnp.int32),
            pltpu.VMEM((CHUNK,), jnp.int32),
            pltpu.VMEM((CHUNK, D), jnp.float32),
            pltpu.VMEM((CHUNK, D), jnp.float32),
            pltpu.VMEM((CHUNK, D), jnp.float32),
            pltpu.SemaphoreType.DMA,
            pltpu.SemaphoreType.DMA,
            pltpu.SemaphoreType.DMA,
            pltpu.SemaphoreType.DMA,
            pltpu.SemaphoreType.DMA,
            pltpu.SemaphoreType.DMA,
        ],
    )
    def _sc_gather(words, pos, neg, emb, oemb, out,
                   iw_v, ip_v, in_v, rw_v, rp_v, rn_v,
                   siw, sip, sin, sw, sp, sn):
        wid = lax.axis_index("s")
        base = wid * CHUNK
        ciw = pltpu.make_async_copy(words.at[pl.ds(base, CHUNK)], iw_v, siw)
        cip = pltpu.make_async_copy(pos.at[pl.ds(base, CHUNK)], ip_v, sip)
        cin = pltpu.make_async_copy(neg.at[pl.ds(base, CHUNK)], in_v, sin)
        ciw.start()
        cip.start()
        cin.start()
        cw = pltpu.make_async_copy(emb.at[iw_v], rw_v, sw)
        cp = pltpu.make_async_copy(oemb.at[ip_v], rp_v, sp)
        cn = pltpu.make_async_copy(oemb.at[in_v], rn_v, sn)
        ciw.wait()
        cw.start()
        cip.wait()
        cp.start()
        cin.wait()
        cn.start()
        cw.wait()
        pltpu.sync_copy(rw_v, out.at[pl.ds(base, CHUNK)])
        cp.wait()
        pltpu.sync_copy(rp_v, out.at[pl.ds(B + base, CHUNK)])
        cn.wait()
        pltpu.sync_copy(rn_v, out.at[pl.ds(2 * B + base, CHUNK)])

    return _sc_gather


def _tc_loss_body(g_ref, out_ref):
    t = g_ref[0:B, :]
    p = g_ref[B:2 * B, :]
    n = g_ref[2 * B:3 * B, :]
    tt = t.T
    s_pos = jnp.sum(tt * p)
    s_neg = jnp.sum(tt * n)
    r = lax.broadcasted_iota(jnp.int32, (8, 128), 0)
    c = lax.broadcasted_iota(jnp.int32, (8, 128), 1)
    ma = ((r == 0) & (c == 0)).astype(jnp.float32)
    mb = ((r == 0) & (c == 1)).astype(jnp.float32)
    v = s_pos * ma - s_neg * mb
    ls = jnp.minimum(v, 0.0) - jnp.log1p(jnp.exp(-jnp.abs(v)))
    out_ref[...] = -jnp.sum(ls * (ma + mb))


def kernel(words, pos_contexts, neg_contexts, emb, out_emb):
    g = _build_sc_gather()(words, pos_contexts, neg_contexts, emb, out_emb)
    loss = pl.pallas_call(
        _tc_loss_body,
        out_shape=jax.ShapeDtypeStruct((), jnp.float32),
        out_specs=pl.BlockSpec(memory_space=pltpu.SMEM),
    )(g)
    return loss


# speedup vs baseline: 1.0500x; 1.0009x over previous
import functools

import jax
import jax.numpy as jnp
from jax import lax
from jax.experimental import pallas as pl
from jax.experimental.pallas import tpu as pltpu
from jax.experimental.pallas import tpu_sc as plsc

B = 128
D = 128
NW = 16
CHUNK = B // NW


@functools.cache
def _build_sc_gather():
    mesh = plsc.VectorSubcoreMesh(
        core_axis_name="c", subcore_axis_name="s", num_cores=1)

    @functools.partial(
        pl.kernel,
        mesh=mesh,
        out_type=jax.ShapeDtypeStruct((3 * B, D), jnp.float32),
        scratch_types=[
            pltpu.VMEM((CHUNK,), jnp.int32),
            pltpu.VMEM((CHUNK,), jnp.int32),
            pltpu.VMEM((CHUNK,), jnp.int32),
            pltpu.VMEM((CHUNK, D), jnp.float32),
            pltpu.VMEM((CHUNK, D), jnp.float32),
            pltpu.VMEM((CHUNK, D), jnp.float32),
            pltpu.SemaphoreType.DMA,
            pltpu.SemaphoreType.DMA,
            pltpu.SemaphoreType.DMA,
            pltpu.SemaphoreType.DMA,
            pltpu.SemaphoreType.DMA,
            pltpu.SemaphoreType.DMA,
        ],
    )
    def _sc_gather(words, pos, neg, emb, oemb, out,
                   iw_v, ip_v, in_v, rw_v, rp_v, rn_v,
                   siw, sip, sin, sw, sp, sn):
        wid = lax.axis_index("s")
        base = wid * CHUNK
        ciw = pltpu.make_async_copy(words.at[pl.ds(base, CHUNK)], iw_v, siw)
        cip = pltpu.make_async_copy(pos.at[pl.ds(base, CHUNK)], ip_v, sip)
        cin = pltpu.make_async_copy(neg.at[pl.ds(base, CHUNK)], in_v, sin)
        ciw.start()
        cip.start()
        cin.start()
        cw = pltpu.make_async_copy(emb.at[iw_v], rw_v, sw)
        cp = pltpu.make_async_copy(oemb.at[ip_v], rp_v, sp)
        cn = pltpu.make_async_copy(oemb.at[in_v], rn_v, sn)
        ciw.wait()
        cw.start()
        cip.wait()
        cp.start()
        cin.wait()
        cn.start()
        cww = pltpu.make_async_copy(rw_v, out.at[pl.ds(base, CHUNK)], siw)
        cpw = pltpu.make_async_copy(rp_v, out.at[pl.ds(B + base, CHUNK)], sip)
        cnw = pltpu.make_async_copy(rn_v, out.at[pl.ds(2 * B + base, CHUNK)],
                                    sin)
        cw.wait()
        cww.start()
        cp.wait()
        cpw.start()
        cn.wait()
        cnw.start()
        cww.wait()
        cpw.wait()
        cnw.wait()

    return _sc_gather


def _tc_loss_body(g_ref, out_ref):
    t = g_ref[0:B, :]
    p = g_ref[B:2 * B, :]
    n = g_ref[2 * B:3 * B, :]
    tt = t.T
    s_pos = jnp.sum(tt * p)
    s_neg = jnp.sum(tt * n)
    r = lax.broadcasted_iota(jnp.int32, (8, 128), 0)
    c = lax.broadcasted_iota(jnp.int32, (8, 128), 1)
    ma = ((r == 0) & (c == 0)).astype(jnp.float32)
    mb = ((r == 0) & (c == 1)).astype(jnp.float32)
    v = s_pos * ma - s_neg * mb
    ls = jnp.minimum(v, 0.0) - jnp.log1p(jnp.exp(-jnp.abs(v)))
    out_ref[...] = -jnp.sum(ls * (ma + mb))


def kernel(words, pos_contexts, neg_contexts, emb, out_emb):
    g = _build_sc_gather()(words, pos_contexts, neg_contexts, emb, out_emb)
    loss = pl.pallas_call(
        _tc_loss_body,
        out_shape=jax.ShapeDtypeStruct((), jnp.float32),
        out_specs=pl.BlockSpec(memory_space=pltpu.SMEM),
    )(g)
    return loss
